# baseline (speedup 1.0000x reference)
import functools

import jax
import jax.numpy as jnp
from jax.experimental import pallas as pl
from jax.experimental.pallas import tpu as pltpu

E = 64
NEG = -3.0e38
FE = float(E)


def _top2_softmax(logits):
    iota = jax.lax.broadcasted_iota(jnp.int32, logits.shape, 1).astype(jnp.float32)
    m1 = jnp.max(logits, axis=1, keepdims=True)
    i1 = jnp.min(jnp.where(logits == m1, iota, FE), axis=1, keepdims=True)
    masked = jnp.where(iota == i1, NEG, logits)
    m2 = jnp.max(masked, axis=1, keepdims=True)
    i2 = jnp.min(jnp.where(masked == m2, iota, FE), axis=1, keepdims=True)
    w2 = 1.0 / (1.0 + jnp.exp(m1 - m2))
    w1 = 1.0 - w2
    w = jnp.concatenate([w1, w2], axis=1)
    i = jnp.concatenate([i1, i2], axis=1).astype(jnp.int32)
    return w, i


def _router_block(x_ref, wt_ref, l_ref, w_ref, i_ref, scratch_ref, nsteps):
    step = pl.program_id(0)

    @pl.when(step > 0)
    def _emit_prev():
        logits = scratch_ref[...]
        l_ref[...] = logits
        w, i = _top2_softmax(logits)
        w_ref[...] = w
        i_ref[...] = i

    @pl.when(step < nsteps - 1)
    def _matmul():
        scratch_ref[...] = jax.lax.dot_general(
            x_ref[...], wt_ref[...], (((1,), (0,)), ((), ())),
            preferred_element_type=jnp.float32,
        )


@functools.partial(jax.jit, static_argnames=("rows",))
def _router(x2d, wt, rows):
    n, d = x2d.shape
    nb = n // rows
    nsteps = nb + 1
    last = nb - 1

    def shift(i):
        j = jnp.maximum(i, 1) - 1
        return (j, 0)

    return pl.pallas_call(
        functools.partial(_router_block, nsteps=nsteps),
        grid=(nsteps,),
        in_specs=[
            pl.BlockSpec((rows, d), lambda i: (jnp.minimum(i, last), 0)),
            pl.BlockSpec((d, E), lambda i: (0, 0)),
        ],
        out_specs=[
            pl.BlockSpec((rows, E), shift),
            pl.BlockSpec((rows, 2), shift),
            pl.BlockSpec((rows, 2), shift),
        ],
        out_shape=[
            jax.ShapeDtypeStruct((n, E), jnp.float32),
            jax.ShapeDtypeStruct((n, 2), jnp.float32),
            jax.ShapeDtypeStruct((n, 2), jnp.int32),
        ],
        scratch_shapes=[pltpu.VMEM((rows, E), jnp.float32)],
    )(x2d, wt)


def kernel(x, W):
    b, t, d = x.shape
    logits, weights, indices = _router(x.reshape(b * t, d), W.T, 1024)
    return (
        weights.reshape(b, t, 2),
        indices.reshape(b, t, 2),
        logits.reshape(b, t, E),
    )


# speedup vs baseline: 1.1262x; 1.1262x over previous
import functools

import jax
import jax.numpy as jnp
from jax.experimental import pallas as pl

E = 64
NEG = -3.0e38
FE = float(E)


def _router_block(x_ref, w_ref_in, l_ref, w_ref, i_ref):
    logits = jax.lax.dot_general(
        x_ref[...], w_ref_in[...], (((1,), (1,)), ((), ())),
        preferred_element_type=jnp.float32,
    )
    l_ref[...] = logits

    iota = jax.lax.broadcasted_iota(jnp.int32, logits.shape, 1).astype(jnp.float32)
    m1 = jnp.max(logits, axis=1, keepdims=True)
    i1 = jnp.min(jnp.where(logits == m1, iota, FE), axis=1, keepdims=True)
    masked = jnp.where(iota == i1, NEG, logits)
    m2 = jnp.max(masked, axis=1, keepdims=True)
    i2 = jnp.min(jnp.where(masked == m2, iota, FE), axis=1, keepdims=True)
    w2 = 1.0 / (1.0 + jnp.exp(m1 - m2))
    w1 = 1.0 - w2

    w_ref[...] = jnp.concatenate([w1, w2], axis=1)
    i_ref[...] = jnp.concatenate([i1, i2], axis=1).astype(jnp.int32)


@functools.partial(jax.jit, static_argnames=("rows",))
def _router(x2d, w, rows):
    n, d = x2d.shape
    grid = (n // rows,)
    return pl.pallas_call(
        _router_block,
        grid=grid,
        in_specs=[
            pl.BlockSpec((rows, d), lambda i: (i, 0)),
            pl.BlockSpec((E, d), lambda i: (0, 0)),
        ],
        out_specs=[
            pl.BlockSpec((rows, E), lambda i: (i, 0)),
            pl.BlockSpec((rows, 2), lambda i: (i, 0)),
            pl.BlockSpec((rows, 2), lambda i: (i, 0)),
        ],
        out_shape=[
            jax.ShapeDtypeStruct((n, E), jnp.float32),
            jax.ShapeDtypeStruct((n, 2), jnp.float32),
            jax.ShapeDtypeStruct((n, 2), jnp.int32),
        ],
    )(x2d, w)


def kernel(x, W):
    b, t, d = x.shape
    logits, weights, indices = _router(x.reshape(b * t, d), W, 1024)
    return (
        weights.reshape(b, t, 2),
        indices.reshape(b, t, 2),
        logits.reshape(b, t, E),
    )
